# Initial kernel scaffold; baseline (speedup 1.0000x reference)
import jax
import jax.numpy as jnp
from jax.experimental import pallas as pl


def kernel(feature_idx, feature_vals, feature_embedding, W1, b1, W2, b2, W3, b3, cw0, cb0, cw1, cb1, cw2, cb2, Wd, bd):
    raise NotImplementedError("write your pallas kernel here")


# speedup vs baseline: 2.5321x; 2.5321x over previous
import functools

import jax
import jax.numpy as jnp
import numpy as np
from jax import lax
from jax.experimental import pallas as pl
from jax.experimental.pallas import tpu as pltpu
from jax.experimental.pallas import tpu_sc as plsc

B = 4096
F = 26
V = 100000
D = 32
IN_DIM = F * D
HID = 400
BF = B * F


_NC = 2
_NS = 16
_NW = _NC * _NS
_B_PER_W = BF // _NW


def _sc_gather(idx_hbm, table_hbm, out_hbm, idx_v, rows_v, sem):
    wid = lax.axis_index("s") * _NC + lax.axis_index("c")
    base = wid * _B_PER_W
    pltpu.sync_copy(idx_hbm.at[pl.ds(base, _B_PER_W)], idx_v)
    pltpu.async_copy(table_hbm.at[idx_v], rows_v, sem).wait()
    pltpu.sync_copy(rows_v, out_hbm.at[pl.ds(base, _B_PER_W)])


def _gather_rows(idx_flat, table):
    mesh = plsc.VectorSubcoreMesh(core_axis_name="c", subcore_axis_name="s")
    k = functools.partial(
        pl.kernel,
        mesh=mesh,
        compiler_params=pltpu.CompilerParams(use_tc_tiling_on_sc=False),
        out_type=jax.ShapeDtypeStruct((BF, D), jnp.float32),
        scratch_types=[
            pltpu.VMEM((_B_PER_W,), jnp.int32),
            pltpu.VMEM((_B_PER_W, D), jnp.float32),
            pltpu.SemaphoreType.DMA,
        ],
    )(_sc_gather)
    return k(idx_flat, table)


_BLK = 512
_NB = B // _BLK


def _dense_body(g_ref, vals_ref, e_ref, w1_ref, b1_ref, w2_ref, b2_ref,
                w3_ref, b3_ref, cw_ref, cb_ref, wdh_ref, wdx_ref, bd_ref,
                out_ref):
    f32 = jnp.float32
    vals = vals_ref[...]
    scale = jnp.dot(vals, e_ref[...], preferred_element_type=f32)
    x0 = g_ref[...] * scale
    h = jnp.maximum(jnp.dot(x0, w1_ref[...], preferred_element_type=f32)
                    + b1_ref[...], 0.0)
    h = jnp.maximum(jnp.dot(h, w2_ref[...], preferred_element_type=f32)
                    + b2_ref[...], 0.0)
    h = jnp.maximum(jnp.dot(h, w3_ref[...], preferred_element_type=f32)
                    + b3_ref[...], 0.0)
    xc = x0
    for i in range(3):
        xw = jnp.sum(xc * cw_ref[i, :][None, :], axis=1, keepdims=True)
        xc = x0 * xw + cb_ref[i, :][None, :] + xc
    logits = (jnp.dot(h, wdh_ref[...], preferred_element_type=f32)
              + jnp.dot(xc, wdx_ref[...], preferred_element_type=f32)
              + bd_ref[...])
    out_ref[...] = jax.nn.sigmoid(logits)


def _dense(gathered, vals, expand, W1, b1, W2, b2, W3, b3, cw, cb, Wdh, Wdx,
           bd):
    full2 = lambda shape: pl.BlockSpec(shape, lambda i: (0, 0))
    return pl.pallas_call(
        _dense_body,
        grid=(_NB,),
        in_specs=[
            pl.BlockSpec((_BLK, IN_DIM), lambda i: (i, 0)),
            pl.BlockSpec((_BLK, F), lambda i: (i, 0)),
            full2((F, IN_DIM)),
            full2((IN_DIM, HID)),
            full2((1, HID)),
            full2((HID, HID)),
            full2((1, HID)),
            full2((HID, HID)),
            full2((1, HID)),
            full2((3, IN_DIM)),
            full2((3, IN_DIM)),
            full2((HID, 1)),
            full2((IN_DIM, 1)),
            full2((1, 1)),
        ],
        out_specs=pl.BlockSpec((_BLK, 1), lambda i: (i, 0)),
        out_shape=jax.ShapeDtypeStruct((B, 1), jnp.float32),
    )(gathered, vals, expand, W1, b1, W2, b2, W3, b3, cw, cb, Wdh, Wdx, bd)


def kernel(feature_idx, feature_vals, feature_embedding, W1, b1, W2, b2, W3,
           b3, cw0, cb0, cw1, cb1, cw2, cb2, Wd, bd):
    gathered = _gather_rows(feature_idx.reshape(BF), feature_embedding)
    gathered = gathered.reshape(B, IN_DIM)
    expand = jnp.asarray(
        np.repeat(np.eye(F, dtype=np.float32), D, axis=1))
    cw = jnp.stack([cw0, cw1, cw2])
    cb = jnp.stack([cb0, cb1, cb2])
    return _dense(gathered, feature_vals, expand, W1, b1.reshape(1, HID),
                  W2, b2.reshape(1, HID), W3, b3.reshape(1, HID), cw, cb,
                  Wd[:HID], Wd[HID:], bd.reshape(1, 1))
